# baseline (speedup 1.0000x reference)
import functools

import jax
import jax.numpy as jnp
from jax import lax
from jax.experimental import pallas as pl
from jax.experimental.pallas import tpu as pltpu
from jax.experimental.pallas import tpu_sc as plsc

N_TAGS = 1000
EMBED = 128
OUT_H = 16
OUT_W = 16
BATCH = 1024
HW = OUT_H * OUT_W


def _sc_gather(table, x):
    info = plsc.get_sparse_core_info()
    nc, ns = 1, info.num_subcores
    nw = nc * ns
    b_per_w = BATCH // nw

    mesh = plsc.VectorSubcoreMesh(
        core_axis_name="c", subcore_axis_name="s", num_cores=1
    )

    @functools.partial(
        pl.kernel,
        mesh=mesh,
        out_type=jax.ShapeDtypeStruct((BATCH, EMBED), jnp.float32),
        scratch_types=[
            pltpu.VMEM((b_per_w,), jnp.int32),
            pltpu.VMEM((b_per_w, EMBED), jnp.float32),
            pltpu.SemaphoreType.DMA,
        ],
    )
    def gather_kernel(table_hbm, idx_hbm, out_hbm, idx_v, rows_v, sem):
        wid = lax.axis_index("s") * nc + lax.axis_index("c")
        base = wid * b_per_w
        pltpu.sync_copy(idx_hbm.at[pl.ds(base, b_per_w)], idx_v)
        pltpu.async_copy(table_hbm.at[idx_v], rows_v, sem).wait()
        pltpu.sync_copy(rows_v, out_hbm.at[pl.ds(base, b_per_w)])

    return gather_kernel(table, x)


def _tc_broadcast(emb, block_b=32):
    def body(emb_ref, out_ref):
        i = pl.program_id(0)
        rows = emb_ref[pl.ds(i * block_b, block_b), :]
        out_ref[...] = jnp.broadcast_to(
            rows[:, None, :], (block_b, HW, EMBED)
        )

    return pl.pallas_call(
        body,
        grid=(BATCH // block_b,),
        in_specs=[pl.BlockSpec((BATCH, EMBED), lambda i: (0, 0))],
        out_specs=pl.BlockSpec((block_b, HW, EMBED), lambda i: (i, 0, 0)),
        out_shape=jax.ShapeDtypeStruct((BATCH, HW, EMBED), jnp.float32),
    )(emb)


def kernel(x, table):
    emb = _sc_gather(table, x)
    out = _tc_broadcast(emb)
    out = out.reshape(BATCH, OUT_H, OUT_W, EMBED)
    return out.transpose(0, 3, 1, 2)


# speedup vs baseline: 1.0393x; 1.0393x over previous
import functools

import jax
import jax.numpy as jnp
from jax import lax
from jax.experimental import pallas as pl
from jax.experimental.pallas import tpu as pltpu
from jax.experimental.pallas import tpu_sc as plsc

N_TAGS = 1000
EMBED = 128
OUT_H = 16
OUT_W = 16
BATCH = 1024
HW = OUT_H * OUT_W
TC_ROWS = 256


def _sc_gather(table, x_hi):
    n_rows = BATCH - TC_ROWS
    info = plsc.get_sparse_core_info()
    ns = info.num_subcores
    b_per_w = n_rows // ns

    mesh = plsc.VectorSubcoreMesh(
        core_axis_name="c", subcore_axis_name="s", num_cores=1
    )

    @functools.partial(
        pl.kernel,
        mesh=mesh,
        out_type=jax.ShapeDtypeStruct((n_rows, EMBED), jnp.float32),
        scratch_types=[
            pltpu.VMEM((b_per_w,), jnp.int32),
            pltpu.VMEM((b_per_w, EMBED), jnp.float32),
            pltpu.SemaphoreType.DMA,
        ],
    )
    def gather_kernel(table_hbm, idx_hbm, out_hbm, idx_v, rows_v, sem):
        wid = lax.axis_index("s")
        base = wid * b_per_w
        pltpu.sync_copy(idx_hbm.at[pl.ds(base, b_per_w)], idx_v)
        pltpu.async_copy(table_hbm.at[idx_v], rows_v, sem).wait()
        pltpu.sync_copy(rows_v, out_hbm.at[pl.ds(base, b_per_w)])

    return gather_kernel(table, x_hi)


def _tc_gather_broadcast_lo(x, table, block_b=32):
    def body(x_ref, table_ref, out_ref):
        i = pl.program_id(0)
        for b in range(block_b):
            idx = x_ref[i * block_b + b]
            row = table_ref[pl.ds(idx, 1), :]
            out_ref[b] = jnp.broadcast_to(row, (HW, EMBED))

    return pl.pallas_call(
        body,
        grid=(TC_ROWS // block_b,),
        in_specs=[
            pl.BlockSpec(memory_space=pltpu.SMEM),
            pl.BlockSpec((N_TAGS, EMBED), lambda i: (0, 0)),
        ],
        out_specs=pl.BlockSpec((block_b, HW, EMBED), lambda i: (i, 0, 0)),
        out_shape=jax.ShapeDtypeStruct((BATCH, HW, EMBED), jnp.float32),
    )(x, table)


def _tc_broadcast_hi(emb_hi, buf, block_b=32):
    n_rows = BATCH - TC_ROWS
    base_blk = TC_ROWS // block_b

    def body(emb_ref, buf_ref, out_ref):
        i = pl.program_id(0)
        rows = emb_ref[pl.ds(i * block_b, block_b), :]
        out_ref[...] = jnp.broadcast_to(rows[:, None, :], (block_b, HW, EMBED))

    return pl.pallas_call(
        body,
        grid=(n_rows // block_b,),
        in_specs=[
            pl.BlockSpec((n_rows, EMBED), lambda i: (0, 0)),
            pl.BlockSpec(memory_space=pl.ANY),
        ],
        out_specs=pl.BlockSpec((block_b, HW, EMBED), lambda i: (base_blk + i, 0, 0)),
        out_shape=jax.ShapeDtypeStruct((BATCH, HW, EMBED), jnp.float32),
        input_output_aliases={1: 0},
    )(emb_hi, buf)


def kernel(x, table):
    emb_hi = _sc_gather(table, x[TC_ROWS:])
    buf = _tc_gather_broadcast_lo(x, table)
    out = _tc_broadcast_hi(emb_hi, buf)
    out = out.reshape(BATCH, OUT_H, OUT_W, EMBED)
    return out.transpose(0, 3, 1, 2)


# speedup vs baseline: 1.0596x; 1.0196x over previous
import functools

import jax
import jax.numpy as jnp
from jax import lax
from jax.experimental import pallas as pl
from jax.experimental.pallas import tpu as pltpu
from jax.experimental.pallas import tpu_sc as plsc

N_TAGS = 1000
EMBED = 128
OUT_H = 16
OUT_W = 16
BATCH = 1024
HW = OUT_H * OUT_W
TC_ROWS = 128


def _sc_gather(table, x):
    n_rows = BATCH - TC_ROWS
    info = plsc.get_sparse_core_info()
    ns = info.num_subcores
    b_per_w = n_rows // ns

    mesh = plsc.VectorSubcoreMesh(
        core_axis_name="c", subcore_axis_name="s", num_cores=1
    )

    @functools.partial(
        pl.kernel,
        mesh=mesh,
        out_type=jax.ShapeDtypeStruct((n_rows, EMBED), jnp.float32),
        scratch_types=[
            pltpu.VMEM((b_per_w,), jnp.int32),
            pltpu.VMEM((b_per_w, EMBED), jnp.float32),
            pltpu.SemaphoreType.DMA,
        ],
    )
    def gather_kernel(table_hbm, idx_hbm, out_hbm, idx_v, rows_v, sem):
        wid = lax.axis_index("s")
        base = wid * b_per_w
        pltpu.sync_copy(idx_hbm.at[pl.ds(TC_ROWS + base, b_per_w)], idx_v)
        pltpu.async_copy(table_hbm.at[idx_v], rows_v, sem).wait()
        pltpu.sync_copy(rows_v, out_hbm.at[pl.ds(base, b_per_w)])

    return gather_kernel(table, x)


def _tc_gather_broadcast_lo(x, table, block_b=32):
    def body(x_ref, table_ref, out_ref):
        i = pl.program_id(0)
        for b in range(block_b):
            idx = x_ref[i * block_b + b]
            row = table_ref[pl.ds(idx, 1), :]
            out_ref[b] = jnp.broadcast_to(row, (HW, EMBED))

    return pl.pallas_call(
        body,
        grid=(TC_ROWS // block_b,),
        in_specs=[
            pl.BlockSpec(memory_space=pltpu.SMEM),
            pl.BlockSpec((N_TAGS, EMBED), lambda i: (0, 0)),
        ],
        out_specs=pl.BlockSpec((block_b, HW, EMBED), lambda i: (i, 0, 0)),
        out_shape=jax.ShapeDtypeStruct((BATCH, HW, EMBED), jnp.float32),
    )(x, table)


def _tc_broadcast_hi(emb_hi, buf, block_b=32):
    n_rows = BATCH - TC_ROWS
    base_blk = TC_ROWS // block_b

    def body(emb_ref, buf_ref, out_ref):
        i = pl.program_id(0)
        rows = emb_ref[pl.ds(i * block_b, block_b), :]
        out_ref[...] = jnp.broadcast_to(rows[:, None, :], (block_b, HW, EMBED))

    return pl.pallas_call(
        body,
        grid=(n_rows // block_b,),
        in_specs=[
            pl.BlockSpec((n_rows, EMBED), lambda i: (0, 0)),
            pl.BlockSpec(memory_space=pl.ANY),
        ],
        out_specs=pl.BlockSpec((block_b, HW, EMBED), lambda i: (base_blk + i, 0, 0)),
        out_shape=jax.ShapeDtypeStruct((BATCH, HW, EMBED), jnp.float32),
        input_output_aliases={1: 0},
    )(emb_hi, buf)


def kernel(x, table):
    emb_hi = _sc_gather(table, x)
    buf = _tc_gather_broadcast_lo(x, table)
    out = _tc_broadcast_hi(emb_hi, buf)
    out = out.reshape(BATCH, OUT_H, OUT_W, EMBED)
    return out.transpose(0, 3, 1, 2)
